# baseline (speedup 1.0000x reference)
import functools

import jax
import jax.numpy as jnp
from jax import lax
from jax.experimental import pallas as pl
from jax.experimental.pallas import tpu as pltpu
from jax.experimental.pallas import tpu_sc as plsc

N = 100000
DIM = 256
DG, DP, DL = 128, 64, 64
R = 128
NB = (N + R - 1) // R
NW = 32
BPW = 25
SLAB = BPW * R
L = 16
NGRP = R // L


def _body(g_h, p_h, l_h, wg_h, wp_h, wl_h, out_h,
          idx_g, idx_p, idx_l, rows0, rows1, wg_v, wp_v, wl_v,
          sem_i, sw0, sw1):
    c = lax.axis_index("c")
    s = lax.axis_index("s")
    w = s * 2 + c
    start = (w * (NB - BPW)) // (NW - 1)
    e0 = start * R

    hs = [
        pltpu.async_copy(g_h.at[pl.ds(e0, SLAB)], idx_g, sem_i),
        pltpu.async_copy(p_h.at[pl.ds(e0, SLAB)], idx_p, sem_i),
        pltpu.async_copy(l_h.at[pl.ds(e0, SLAB)], idx_l, sem_i),
        pltpu.async_copy(wg_h, wg_v, sem_i),
        pltpu.async_copy(wp_h, wp_v, sem_i),
        pltpu.async_copy(wl_h, wl_v, sem_i),
    ]
    for h in hs:
        h.wait()

    iota = lax.iota(jnp.int32, L)
    rowv = [k * L + iota for k in range(NGRP)]

    bufs = (rows0, rows1)
    sws = (sw0, sw1)

    def fill_block(j25):
        buf = bufs[j25 % 2]

        def half(tab, idx_ref, col0, ncol, unroll):
            ivs = [idx_ref[pl.ds(j25 * R + k * L, L)] for k in range(NGRP)]

            def col(j, carry):
                jc = j * unroll
                for u in range(unroll):
                    jv = jnp.full((L,), jc + u + col0, jnp.int32)
                    vs = [plsc.load_gather(tab, [ivs[k], jv - col0])
                          for k in range(NGRP)]
                    for k in range(NGRP):
                        plsc.store_scatter(buf, [rowv[k], jv], vs[k])
                return carry

            lax.fori_loop(0, ncol // unroll, col, 0)

        half(wg_v, idx_g, 0, DG, 2)
        half(wp_v, idx_p, DG, DP, 2)
        half(wl_v, idx_l, DG + DP, DL, 2)

    def fire_write(j):
        slot = j % 2
        base = jnp.minimum((start + j) * R, N - R)
        return pltpu.async_copy(bufs[slot], out_h.at[pl.ds(base, R), :],
                                sws[slot])

    wh = [None] * BPW
    for j in range(BPW):
        if j >= 2:
            wh[j - 2].wait()
        fill_block(j)
        wh[j] = fire_write(j)
    wh[BPW - 2].wait()
    wh[BPW - 1].wait()


@jax.jit
def kernel(group, period, ls, Wg, Wp, Wl):
    def layout(x):
        x = x.astype(jnp.int32)
        return jnp.concatenate([x[:(NB - 1) * R], x[N - R:]])

    g1 = layout(group)
    p1 = layout(period)
    l1 = layout(ls)

    mesh = plsc.VectorSubcoreMesh(core_axis_name="c", subcore_axis_name="s")
    run = functools.partial(
        pl.kernel,
        mesh=mesh,
        compiler_params=pltpu.CompilerParams(needs_layout_passes=False),
        out_type=jax.ShapeDtypeStruct((N, DIM), jnp.float32),
        scratch_types=[
            pltpu.VMEM((SLAB,), jnp.int32),
            pltpu.VMEM((SLAB,), jnp.int32),
            pltpu.VMEM((SLAB,), jnp.int32),
            pltpu.VMEM((R, DIM), jnp.float32),
            pltpu.VMEM((R, DIM), jnp.float32),
            pltpu.VMEM((18, DG), jnp.float32),
            pltpu.VMEM((7, DP), jnp.float32),
            pltpu.VMEM((3, DL), jnp.float32),
            pltpu.SemaphoreType.DMA,
            pltpu.SemaphoreType.DMA,
            pltpu.SemaphoreType.DMA,
        ],
    )(_body)
    return run(g1, p1, l1, Wg, Wp, Wl)


# speedup vs baseline: 3.7013x; 3.7013x over previous
import functools

import jax
import jax.numpy as jnp
from jax import lax
from jax.experimental import pallas as pl
from jax.experimental.pallas import tpu as pltpu
from jax.experimental.pallas import tpu_sc as plsc

N = 100000
DIM = 256
DG, DP, DL = 128, 64, 64
R = 128
NB = (N + R - 1) // R
NW = 32
BPW = 25
SLAB = BPW * R
L = 16
NGRP = R // L


def _body(g_h, p_h, l_h, wg_h, wp_h, wl_h, out_h,
          idx_g, idx_p, idx_l, rows0, rows1, wg_v, wp_v, wl_v,
          sem_i, sw0, sw1):
    c = lax.axis_index("c")
    s = lax.axis_index("s")
    w = s * 2 + c
    start = (w * (NB - BPW)) // (NW - 1)
    e0 = start * R

    hs = [
        pltpu.async_copy(g_h.at[pl.ds(e0, SLAB)], idx_g, sem_i),
        pltpu.async_copy(p_h.at[pl.ds(e0, SLAB)], idx_p, sem_i),
        pltpu.async_copy(l_h.at[pl.ds(e0, SLAB)], idx_l, sem_i),
        pltpu.async_copy(wg_h, wg_v, sem_i),
        pltpu.async_copy(wp_h, wp_v, sem_i),
        pltpu.async_copy(wl_h, wl_v, sem_i),
    ]
    for h in hs:
        h.wait()

    iota = lax.iota(jnp.int32, L)
    colc = [m * L + iota for m in range(NGRP)]

    bufs = (rows0, rows1)
    sws = (sw0, sw1)

    def fill_block(j25):
        buf = bufs[j25 % 2]
        base = jnp.full((L,), j25 * R, jnp.int32)

        def node(n, carry):
            nv = base + n
            gb = plsc.load_gather(idx_g, [nv])
            pb = plsc.load_gather(idx_p, [nv])
            lb = plsc.load_gather(idx_l, [nv])
            for m in range(DG // L):
                v = plsc.load_gather(wg_v, [gb, colc[m]])
                buf[n, pl.ds(m * L, L)] = v
            for m in range(DP // L):
                v = plsc.load_gather(wp_v, [pb, colc[m]])
                buf[n, pl.ds(DG + m * L, L)] = v
            for m in range(DL // L):
                v = plsc.load_gather(wl_v, [lb, colc[m]])
                buf[n, pl.ds(DG + DP + m * L, L)] = v
            return carry

        lax.fori_loop(0, R, node, 0)

    def fire_write(j):
        slot = j % 2
        base = jnp.minimum((start + j) * R, N - R)
        return pltpu.async_copy(bufs[slot], out_h.at[pl.ds(base, R), :],
                                sws[slot])

    wh = [None] * BPW
    for j in range(BPW):
        if j >= 2:
            wh[j - 2].wait()
        fill_block(j)
        wh[j] = fire_write(j)
    wh[BPW - 2].wait()
    wh[BPW - 1].wait()


@jax.jit
def kernel(group, period, ls, Wg, Wp, Wl):
    def layout(x):
        x = x.astype(jnp.int32)
        return jnp.concatenate([x[:(NB - 1) * R], x[N - R:]])

    g1 = layout(group)
    p1 = layout(period)
    l1 = layout(ls)

    mesh = plsc.VectorSubcoreMesh(core_axis_name="c", subcore_axis_name="s")
    run = functools.partial(
        pl.kernel,
        mesh=mesh,
        compiler_params=pltpu.CompilerParams(needs_layout_passes=False),
        out_type=jax.ShapeDtypeStruct((N, DIM), jnp.float32),
        scratch_types=[
            pltpu.VMEM((SLAB,), jnp.int32),
            pltpu.VMEM((SLAB,), jnp.int32),
            pltpu.VMEM((SLAB,), jnp.int32),
            pltpu.VMEM((R, DIM), jnp.float32),
            pltpu.VMEM((R, DIM), jnp.float32),
            pltpu.VMEM((18, DG), jnp.float32),
            pltpu.VMEM((7, DP), jnp.float32),
            pltpu.VMEM((3, DL), jnp.float32),
            pltpu.SemaphoreType.DMA,
            pltpu.SemaphoreType.DMA,
            pltpu.SemaphoreType.DMA,
        ],
    )(_body)
    return run(g1, p1, l1, Wg, Wp, Wl)


# speedup vs baseline: 8.5938x; 2.3218x over previous
import functools

import jax
import jax.numpy as jnp
from jax import lax
from jax.experimental import pallas as pl
from jax.experimental.pallas import tpu as pltpu
from jax.experimental.pallas import tpu_sc as plsc

N = 100000
DIM = 256
DG, DP, DL = 128, 64, 64
R = 128
NB = (N + R - 1) // R
NW = 32
BPW = 25
SLAB = BPW * R
L = 16
NGRP = R // L


def _body(g_h, p_h, l_h, wg_h, wp_h, wl_h, out_h,
          idx_g, idx_p, idx_l, rows0, rows1, wg_v, wp_v, wl_v,
          sem_i, sw0, sw1):
    c = lax.axis_index("c")
    s = lax.axis_index("s")
    w = s * 2 + c
    start = (w * (NB - BPW)) // (NW - 1)
    e0 = start * R

    hs = [
        pltpu.async_copy(g_h.at[pl.ds(e0, SLAB)], idx_g, sem_i),
        pltpu.async_copy(p_h.at[pl.ds(e0, SLAB)], idx_p, sem_i),
        pltpu.async_copy(l_h.at[pl.ds(e0, SLAB)], idx_l, sem_i),
        pltpu.async_copy(wg_h, wg_v, sem_i),
        pltpu.async_copy(wp_h, wp_v, sem_i),
        pltpu.async_copy(wl_h, wl_v, sem_i),
    ]
    for h in hs:
        h.wait()

    iota = lax.iota(jnp.int32, L)
    colc = [m * L + iota for m in range(NGRP)]

    bufs = (rows0, rows1)
    sws = (sw0, sw1)

    def fill_block(j25):
        buf = bufs[j25 % 2]
        base = jnp.full((L,), j25 * R, jnp.int32)

        def node(n, carry):
            nv = base + n
            gb = plsc.load_gather(idx_g, [nv])
            pb = plsc.load_gather(idx_p, [nv])
            lb = plsc.load_gather(idx_l, [nv])
            vsg = [plsc.load_gather(wg_v, [gb, colc[m]])
                   for m in range(DG // L)]
            vsp = [plsc.load_gather(wp_v, [pb, colc[m]])
                   for m in range(DP // L)]
            vsl = [plsc.load_gather(wl_v, [lb, colc[m]])
                   for m in range(DL // L)]
            for m in range(DG // L):
                buf[n, pl.ds(m * L, L)] = vsg[m]
            for m in range(DP // L):
                buf[n, pl.ds(DG + m * L, L)] = vsp[m]
            for m in range(DL // L):
                buf[n, pl.ds(DG + DP + m * L, L)] = vsl[m]
            return carry

        lax.fori_loop(0, R, node, 0)

    def fire_write(j):
        slot = j % 2
        base = jnp.minimum((start + j) * R, N - R)
        return pltpu.async_copy(bufs[slot], out_h.at[pl.ds(base, R), :],
                                sws[slot])

    wh = [None] * BPW
    for j in range(BPW):
        if j >= 2:
            wh[j - 2].wait()
        fill_block(j)
        wh[j] = fire_write(j)
    wh[BPW - 2].wait()
    wh[BPW - 1].wait()


@jax.jit
def kernel(group, period, ls, Wg, Wp, Wl):
    def layout(x):
        x = x.astype(jnp.int32)
        return jnp.concatenate([x[:(NB - 1) * R], x[N - R:]])

    g1 = layout(group)
    p1 = layout(period)
    l1 = layout(ls)

    mesh = plsc.VectorSubcoreMesh(core_axis_name="c", subcore_axis_name="s")
    run = functools.partial(
        pl.kernel,
        mesh=mesh,
        compiler_params=pltpu.CompilerParams(needs_layout_passes=False),
        out_type=jax.ShapeDtypeStruct((N, DIM), jnp.float32),
        scratch_types=[
            pltpu.VMEM((SLAB,), jnp.int32),
            pltpu.VMEM((SLAB,), jnp.int32),
            pltpu.VMEM((SLAB,), jnp.int32),
            pltpu.VMEM((R, DIM), jnp.float32),
            pltpu.VMEM((R, DIM), jnp.float32),
            pltpu.VMEM((18, DG), jnp.float32),
            pltpu.VMEM((7, DP), jnp.float32),
            pltpu.VMEM((3, DL), jnp.float32),
            pltpu.SemaphoreType.DMA,
            pltpu.SemaphoreType.DMA,
            pltpu.SemaphoreType.DMA,
        ],
    )(_body)
    return run(g1, p1, l1, Wg, Wp, Wl)
